# baseline (speedup 1.0000x reference)
import functools

import jax
import jax.numpy as jnp
from jax import lax
from jax.experimental import pallas as pl
from jax.experimental.pallas import tpu as pltpu
from jax.experimental.pallas import tpu_sc as plsc

N = 8192
F = 64
O = 512
X, Y, D = 64, 64, 132

NPAIR = D // 2
JW = 40
TROWS = JW * JW * NPAIR
GPP = 16
RPP = 3 * GPP
WPP = 6 * GPP

NC, NS = 2, 16
NW = NC * NS
PPW = N // NW
CHUNK = 8
NG = CHUNK * RPP // 128
NCHUNK = PPW // CHUNK

_PAD = 24.0
_D0 = 425.0
_DSCALE = 128.0 / (905.0 - 425.0)


def _prep_body(pts_ref, idx_ref, w_ref):
    pts = pts_ref[...]
    px = pts[:, 0:1] + _PAD
    py = pts[:, 1:2] + _PAD
    pz = (pts[:, 2:3] - _D0) * _DSCALE
    bx = jnp.floor(px)
    by = jnp.floor(py)
    bz = jnp.floor(pz)
    bxi = bx.astype(jnp.int32)
    byi = by.astype(jnp.int32)
    bzi = bz.astype(jnp.int32)
    q0 = (bzi - 1) // 2

    t64 = lax.broadcasted_iota(jnp.int32, (N, 64), 1)
    oi = (t64 // 16 - 1).astype(jnp.float32)
    oj = ((t64 // 4) % 4 - 1).astype(jnp.float32)
    ok = (t64 % 4 - 1).astype(jnp.float32)
    sim64 = (bx + oi) * px + (by + oj) * py + (bz + ok) * pz
    s = jnp.sum(sim64, axis=1, keepdims=True)

    t96 = lax.broadcasted_iota(jnp.int32, (N, WPP), 1)
    g = t96 // 6
    m = t96 % 6
    oa = (g // 4 - 1).astype(jnp.float32)
    ob = (g % 4 - 1).astype(jnp.float32)
    km = 2 * q0 + m
    kmf = km.astype(jnp.float32)
    valid = (km >= bzi - 1) & (km <= bzi + 2)
    sim96 = (bx + oa) * px + (by + ob) * py + kmf * pz
    w_ref[...] = jnp.where(valid, sim96, 0.0) / s

    t48 = lax.broadcasted_iota(jnp.int32, (N, RPP), 1)
    gg = t48 // 3
    pp = t48 % 3
    ii = bxi + (gg // 4 - 1)
    jj = byi + (gg % 4 - 1)
    idx_ref[...] = ((ii - 23) * JW + (jj - 23)) * NPAIR + q0 + pp


_prep = pl.pallas_call(
    _prep_body,
    out_shape=(
        jax.ShapeDtypeStruct((N, RPP), jnp.int32),
        jax.ShapeDtypeStruct((N, WPP), jnp.float32),
    ),
)


def _pairpack(feature_volume):
    a = feature_volume[23:63, 23:63]
    return jnp.concatenate([a[:, :, 0::2], a[:, :, 1::2]], axis=-1).reshape(
        TROWS, 2 * F)


def _sc_mix_body(idx_hbm, w_hbm, table_hbm, out_hbm, idx_v, vals_v, w_v, out_v, sem):
    wid = lax.axis_index("s") * NC + lax.axis_index("c")

    def chunk_body(c, carry):
        p0 = wid * PPW + c * CHUNK
        pltpu.sync_copy(idx_hbm.at[pl.ds(p0 * RPP, CHUNK * RPP)], idx_v)
        pltpu.sync_copy(w_hbm.at[pl.ds(p0 * WPP, CHUNK * WPP)],
                        w_v.at[pl.ds(0, CHUNK * WPP)])
        copies = [
            pltpu.async_copy(
                table_hbm.at[idx_v.at[pl.ds(gi * 128, 128)]],
                vals_v.at[pl.ds(gi * 128, 128)],
                sem,
            )
            for gi in range(NG)
        ]
        for cp in copies:
            cp.wait()
        for p in range(CHUNK):
            def rstep(r, accs):
                wa = w_v[pl.ds(p * WPP + 2 * r, 16)]
                wka = jnp.full((16,), wa[0], jnp.float32)
                row = p * RPP + r
                acc = tuple(
                    accs[f] + wka * vals_v[row, pl.ds(f * 16, 16)]
                    for f in range(4)
                )
                wb = w_v[pl.ds(p * WPP + 2 * r + 1, 16)]
                wkb = jnp.full((16,), wb[0], jnp.float32)
                return tuple(
                    acc[f] + wkb * vals_v[row, pl.ds(F + f * 16, 16)]
                    for f in range(4)
                )

            z = jnp.zeros((16,), jnp.float32)
            acc = lax.fori_loop(0, RPP, rstep, (z, z, z, z))
            for f in range(4):
                out_v[p, pl.ds(f * 16, 16)] = acc[f]
        pltpu.sync_copy(out_v, out_hbm.at[pl.ds(p0, CHUNK)])
        return carry

    lax.fori_loop(0, NCHUNK, chunk_body, 0)


@functools.lru_cache(maxsize=1)
def _sc_mix():
    return pl.kernel(
        _sc_mix_body,
        out_type=jax.ShapeDtypeStruct((N, F), jnp.float32),
        mesh=plsc.VectorSubcoreMesh(core_axis_name="c", subcore_axis_name="s"),
        scratch_types=[
            pltpu.VMEM((CHUNK * RPP,), jnp.int32),
            pltpu.VMEM((CHUNK * RPP, 2 * F), jnp.float32),
            pltpu.VMEM((CHUNK * WPP + 16,), jnp.float32),
            pltpu.VMEM((CHUNK, F), jnp.float32),
            pltpu.SemaphoreType.DMA,
        ],
    )


_BM = 1024


def _final_body(m_ref, wt_ref, b_ref, o_ref):
    m = m_ref[...]
    mn = m / jnp.sum(m, axis=1, keepdims=True)
    o_ref[...] = (
        lax.dot_general(mn, wt_ref[...], (((1,), (1,)), ((), ())),
                        preferred_element_type=jnp.float32)
        + b_ref[...]
    )


_final = pl.pallas_call(
    _final_body,
    grid=(N // _BM,),
    in_specs=[
        pl.BlockSpec((_BM, F), lambda i: (i, 0)),
        pl.BlockSpec((O, F), lambda i: (0, 0)),
        pl.BlockSpec((1, O), lambda i: (0, 0)),
    ],
    out_specs=pl.BlockSpec((_BM, O), lambda i: (i, 0)),
    out_shape=jax.ShapeDtypeStruct((N, O), jnp.float32),
)


def kernel(sampled_points, feature_volume, W, b):
    ridx, w = _prep(sampled_points)
    table = _pairpack(feature_volume)
    mixed = _sc_mix()(ridx.reshape(N * RPP), w.reshape(N * WPP), table)
    return _final(mixed, W, b.reshape(1, O))
